# baseline (speedup 1.0000x reference)
import jax
import jax.numpy as jnp
from jax.experimental import pallas as pl
from jax.experimental.pallas import tpu as pltpu

B, T, D = 4, 4096, 2048
N, L, DIN = 16, 256, 1024
KB = T // L


def _body(bref, tref, emb, feats, w_ref, bias_ref, out,
          fbuf, pbuf, copy_sem, fsem, psem):
    masks = [jnp.int32(0)] * B
    segs = []
    for n in range(N):
        bn = bref[n]
        kn = tref[n] // L
        segs.append((bn, pl.multiple_of(tref[n], L)))
        bit = jnp.int32(1) << kn
        for b in range(B):
            masks[b] = jnp.where(bn == b, masks[b] | bit, masks[b])

    for b in range(B):
        for k in range(KB):
            uncov = (masks[b] >> k) & 1 == 0

            @pl.when(uncov)
            def _(b=b, k=k):
                pltpu.make_async_copy(
                    emb.at[b, pl.ds(k * L, L), :],
                    out.at[b, pl.ds(k * L, L), :],
                    copy_sem,
                ).start()

    def feats_in(n, slot):
        return pltpu.make_async_copy(feats.at[n], fbuf.at[slot], fsem.at[slot])

    def proj_out(n, slot):
        bn, tn = segs[n]
        return pltpu.make_async_copy(
            pbuf.at[slot], out.at[bn, pl.ds(tn, L), :], psem.at[slot])

    feats_in(0, 0).start()
    for n in range(N):
        slot = n % 2
        if n + 1 < N:
            feats_in(n + 1, (n + 1) % 2).start()
        feats_in(n, slot).wait()
        if n >= 2:
            proj_out(n - 2, slot).wait()
        pbuf[slot] = jnp.dot(fbuf[slot], w_ref[...],
                             preferred_element_type=jnp.float32) + bias_ref[...]
        proj_out(n, slot).start()
    proj_out(N - 2, 0).wait()
    proj_out(N - 1, 1).wait()

    for b in range(B):
        for k in range(KB):
            uncov = (masks[b] >> k) & 1 == 0

            @pl.when(uncov)
            def _(b=b, k=k):
                pltpu.make_async_copy(
                    emb.at[b, pl.ds(k * L, L), :],
                    out.at[b, pl.ds(k * L, L), :],
                    copy_sem,
                ).wait()


def kernel(emb, feats, batch_idxs, time_idxs, W, b):
    b2 = b.reshape(1, D)
    out = pl.pallas_call(
        _body,
        in_specs=[
            pl.BlockSpec(memory_space=pltpu.SMEM),
            pl.BlockSpec(memory_space=pltpu.SMEM),
            pl.BlockSpec(memory_space=pl.ANY),
            pl.BlockSpec(memory_space=pl.ANY),
            pl.BlockSpec(memory_space=pltpu.MemorySpace.VMEM),
            pl.BlockSpec(memory_space=pltpu.MemorySpace.VMEM),
        ],
        out_specs=pl.BlockSpec(memory_space=pl.ANY),
        out_shape=jax.ShapeDtypeStruct((B, T, D), jnp.float32),
        scratch_shapes=[
            pltpu.VMEM((2, L, DIN), jnp.float32),
            pltpu.VMEM((2, L, D), jnp.float32),
            pltpu.SemaphoreType.DMA,
            pltpu.SemaphoreType.DMA((2,)),
            pltpu.SemaphoreType.DMA((2,)),
        ],
    )(batch_idxs, time_idxs, emb, feats, W, b2)
    return out


# speedup vs baseline: 25.5118x; 25.5118x over previous
import jax
import jax.numpy as jnp
from jax.experimental import pallas as pl
from jax.experimental.pallas import tpu as pltpu

B, T, D = 4, 4096, 2048
N, L, DIN = 16, 256, 1024
KB = T // L


def _seg_match(bi, ti, bref, tref):
    covered = None
    seg = jnp.int32(0)
    for n in range(N):
        hit = (bref[n] == bi) & (tref[n] == ti * L)
        seg = jnp.where(hit, jnp.int32(n), seg)
        covered = hit if covered is None else (covered | hit)
    return covered, seg


def _feats_index(bi, ti, bref, tref):
    _, seg = _seg_match(bi, ti, bref, tref)
    return seg, 0, 0


def _emb_index(bi, ti, bref, tref):
    covered, _ = _seg_match(bi, ti, bref, tref)
    lin = bi * KB + ti
    lin = jnp.where(covered, jnp.maximum(lin - 1, 0), lin)
    return lin // KB, lin % KB, 0


def _body(bref, tref, emb_ref, feats_ref, w_ref, b_ref, out_ref):
    bi = pl.program_id(0)
    ti = pl.program_id(1)
    covered, _ = _seg_match(bi, ti, bref, tref)

    @pl.when(covered)
    def _():
        acc = jnp.dot(feats_ref[0].astype(jnp.bfloat16), w_ref[...],
                      preferred_element_type=jnp.float32)
        out_ref[0] = acc + b_ref[...]

    @pl.when(jnp.logical_not(covered))
    def _():
        out_ref[...] = emb_ref[...]


def kernel(emb, feats, batch_idxs, time_idxs, W, b):
    b2 = b.reshape(1, D)
    w16 = W.astype(jnp.bfloat16)
    grid = (B, KB)
    out = pl.pallas_call(
        _body,
        grid_spec=pltpu.PrefetchScalarGridSpec(
            num_scalar_prefetch=2,
            grid=grid,
            in_specs=[
                pl.BlockSpec((1, L, D), _emb_index),
                pl.BlockSpec((1, L, DIN), _feats_index),
                pl.BlockSpec((DIN, D), lambda bi, ti, bref, tref: (0, 0)),
                pl.BlockSpec((1, D), lambda bi, ti, bref, tref: (0, 0)),
            ],
            out_specs=pl.BlockSpec((1, L, D),
                                   lambda bi, ti, bref, tref: (bi, ti, 0)),
        ),
        out_shape=jax.ShapeDtypeStruct((B, T, D), jnp.float32),
    )(batch_idxs, time_idxs, emb, feats, w16, b2)
    return out


# speedup vs baseline: 30.7499x; 1.2053x over previous
import jax
import jax.numpy as jnp
from jax.experimental import pallas as pl
from jax.experimental.pallas import tpu as pltpu

B, T, D = 4, 4096, 2048
N, L, DIN = 16, 256, 1024
BT = 1024
S = BT // L


def _slot_match(bi, ti, s, bref, tref):
    covered = None
    seg = jnp.int32(0)
    for n in range(N):
        hit = (bref[n] == bi) & (tref[n] == ti * BT + s * L)
        seg = jnp.where(hit, jnp.int32(n), seg)
        covered = hit if covered is None else (covered | hit)
    return covered, seg


def _feats_index(s):
    def index(bi, ti, bref, tref):
        _, seg = _slot_match(bi, ti, s, bref, tref)
        return seg, 0, 0
    return index


def _body(bref, tref, emb_ref, f0, f1, f2, f3, w_ref, b_ref, out_ref):
    bi = pl.program_id(0)
    ti = pl.program_id(1)
    frefs = (f0, f1, f2, f3)
    for s in range(S):
        covered, _ = _slot_match(bi, ti, s, bref, tref)

        @pl.when(covered)
        def _(s=s):
            acc = jnp.dot(frefs[s][0].astype(jnp.bfloat16), w_ref[...],
                          preferred_element_type=jnp.float32)
            out_ref[0, s * L:(s + 1) * L, :] = acc + b_ref[...]

        @pl.when(jnp.logical_not(covered))
        def _(s=s):
            out_ref[0, s * L:(s + 1) * L, :] = emb_ref[0, s * L:(s + 1) * L, :]


def kernel(emb, feats, batch_idxs, time_idxs, W, b):
    b2 = b.reshape(1, D)
    w16 = W.astype(jnp.bfloat16)
    grid = (B, T // BT)
    feats_specs = [pl.BlockSpec((1, L, DIN), _feats_index(s))
                   for s in range(S)]
    out = pl.pallas_call(
        _body,
        grid_spec=pltpu.PrefetchScalarGridSpec(
            num_scalar_prefetch=2,
            grid=grid,
            in_specs=[
                pl.BlockSpec((1, BT, D), lambda bi, ti, bref, tref: (bi, ti, 0)),
                *feats_specs,
                pl.BlockSpec((DIN, D), lambda bi, ti, bref, tref: (0, 0)),
                pl.BlockSpec((1, D), lambda bi, ti, bref, tref: (0, 0)),
            ],
            out_specs=pl.BlockSpec((1, BT, D),
                                   lambda bi, ti, bref, tref: (bi, ti, 0)),
        ),
        out_shape=jax.ShapeDtypeStruct((B, T, D), jnp.float32),
    )(batch_idxs, time_idxs, emb, feats, feats, feats, feats, w16, b2)
    return out
